# baseline (device time: 183718 ns/iter reference)
import numpy as np

import jax
import jax.numpy as jnp
from jax import lax
from jax.experimental import pallas as pl
from jax.experimental.pallas import tpu as pltpu

N_DEV = 8
B = 2
SQ = 512
S_GLOBAL = N_DEV * SQ
D = 1024
HQ = 8
DH = 128
SCALE = 0.08838834764831843


def _rope(t, cos, sin):
    idx = lax.broadcasted_iota(jnp.int32, t.shape, 1)
    t_r = jnp.where(idx % 2 == 0,
                    -jnp.roll(t, -1, axis=1),
                    jnp.roll(t, 1, axis=1))
    return t * cos + t_r * sin


_NEXT = [1, 2, 3, 7, 0, 4, 5, 6]
_PREV = [4, 0, 1, 2, 5, 6, 7, 3]


def kernel(x, Wq, Wk, Wv, Wo):
    xb = x.astype(jnp.bfloat16)
    wqb = Wq.astype(jnp.bfloat16)
    wkb = Wk.astype(jnp.bfloat16)
    wvb = Wv.astype(jnp.bfloat16)
    wob = Wo.astype(jnp.bfloat16)

    inv = 1.0 / (10000.0 ** (np.arange(0, DH, 2) / DH))
    pos = (jnp.arange(S_GLOBAL, dtype=jnp.float32)[:, None]
           * jnp.asarray(inv, dtype=jnp.float32)[None, :])
    cos = jnp.repeat(jnp.cos(pos), 2, axis=-1).astype(jnp.bfloat16)
    sin = jnp.repeat(jnp.sin(pos), 2, axis=-1).astype(jnp.bfloat16)

    my_id = lax.axis_index("i")
    nxt_t = jnp.asarray(_NEXT, dtype=jnp.int32)
    prv_t = jnp.asarray(_PREV, dtype=jnp.int32)
    o_list = [my_id.astype(jnp.int32)]
    for _ in range(4):
        o_list.append(prv_t[o_list[-1]])
    n_list = [nxt_t[my_id]]
    for _ in range(2):
        n_list.append(nxt_t[n_list[-1]])
    ring = jnp.stack([nxt_t[my_id], prv_t[my_id]] + o_list + n_list)

    def body(ring_ref, x_ref, wq_ref, wk_ref, wv_ref, wo_ref, cos_ref,
             sin_ref, out_ref, x_all, q_all, k_scr, v_scr, acc_ref,
             ctx_ref, send_sems_r, recv_sems_r, send_sems_l, recv_sems_l):
        right = ring_ref[0]
        left = ring_ref[1]

        barrier_sem = pltpu.get_barrier_semaphore()
        for nbr in (left, right):
            pl.semaphore_signal(barrier_sem, inc=1, device_id=(nbr,),
                                device_id_type=pl.DeviceIdType.MESH)
        pl.semaphore_wait(barrier_sem, 2)

        my = ring_ref[2]

        x_all[pl.ds(my, 1)] = x_ref[:][None]

        def make_rdma(t, slot, target, s_sems, r_sems):
            return pltpu.make_async_remote_copy(
                src_ref=x_all.at[slot],
                dst_ref=x_all.at[slot],
                send_sem=s_sems.at[t],
                recv_sem=r_sems.at[t],
                device_id=(target,),
                device_id_type=pl.DeviceIdType.MESH,
            )

        def mk_r(t, slot):
            return make_rdma(t, slot, right, send_sems_r, recv_sems_r)

        def mk_l(t, slot):
            return make_rdma(t, slot, left, send_sems_l, recv_sems_l)

        r0 = mk_r(0, my)
        r0.start()
        l0 = mk_l(0, my)
        l0.start()

        cos_q = cos_ref[pl.ds(my * SQ, SQ), :].astype(jnp.float32)
        sin_q = sin_ref[pl.ds(my * SQ, SQ), :].astype(jnp.float32)
        for b in range(B):
            qf = jnp.dot(x_ref[b], wq_ref[...],
                         preferred_element_type=jnp.float32)
            for h in range(HQ):
                hs = slice(h * DH, (h + 1) * DH)
                q_all[b, h] = (_rope(qf[:, hs], cos_q, sin_q)
                               * SCALE).astype(jnp.bfloat16)
        v_scr[...] = (lax.broadcasted_iota(
            jnp.int32, (HQ, B, SQ, 2 * DH), 3) == DH).astype(jnp.bfloat16)
        acc_ref[...] = jnp.zeros((B, HQ, SQ, 2 * DH), dtype=jnp.float32)

        def process(o):
            cos_o = cos_ref[pl.ds(o * SQ, SQ), :].astype(jnp.float32)
            sin_o = sin_ref[pl.ds(o * SQ, SQ), :].astype(jnp.float32)

            for b in range(B):
                xo = x_all[o, b]
                kf = jnp.dot(xo, wk_ref[...],
                             preferred_element_type=jnp.float32)
                for h in range(HQ):
                    hs = slice(h * DH, (h + 1) * DH)
                    k_scr[h, b] = _rope(kf[:, hs], cos_o, sin_o).astype(
                        jnp.bfloat16)
                vf = jnp.dot(xo, wv_ref[...],
                             preferred_element_type=jnp.float32)
                for h in range(HQ):
                    hs = slice(h * DH, (h + 1) * DH)
                    v_scr[h, b, :, :DH] = vf[:, hs].astype(jnp.bfloat16)

            def bh_body(i, _):
                b = i // HQ
                h = i - b * HQ
                s = lax.dot_general(
                    q_all[b, h], k_scr[h, b], (((1,), (1,)), ((), ())),
                    preferred_element_type=jnp.float32)
                p = jnp.exp(s).astype(jnp.bfloat16)
                acc_ref[b, h] = acc_ref[b, h] + jnp.dot(
                    p, v_scr[h, b],
                    preferred_element_type=jnp.float32)
                return 0

            lax.fori_loop(0, B * HQ, bh_body, 0)

        process(my)
        r0.wait()
        r1 = mk_r(1, ring_ref[3])
        r1.start()
        process(ring_ref[3])
        l0.wait()
        l1 = mk_l(1, ring_ref[7])
        l1.start()
        process(ring_ref[7])
        r1.wait()
        r2 = mk_r(2, ring_ref[4])
        r2.start()
        process(ring_ref[4])
        l1.wait()
        l2 = mk_l(2, ring_ref[8])
        l2.start()
        process(ring_ref[8])
        r2.wait()
        r3 = mk_r(3, ring_ref[5])
        r3.start()
        process(ring_ref[5])
        l2.wait()
        process(ring_ref[9])
        r3.wait()
        process(ring_ref[6])

        for b in range(B):
            for h in range(HQ):
                a = acc_ref[b, h]
                ctx_ref[b * SQ:(b + 1) * SQ, h * DH:(h + 1) * DH] = (
                    a[:, :DH] / a[:, DH:DH + 1]).astype(jnp.bfloat16)
        out = jnp.dot(ctx_ref[...], wo_ref[...],
                      preferred_element_type=jnp.float32)
        out_ref[...] = out.reshape(B, SQ, D)

    return pl.pallas_call(
        body,
        out_shape=jax.ShapeDtypeStruct((B, SQ, D), jnp.float32),
        in_specs=[pl.BlockSpec(memory_space=pltpu.SMEM)]
        + [pl.BlockSpec(memory_space=pltpu.VMEM)] * 7,
        out_specs=pl.BlockSpec(memory_space=pltpu.VMEM),
        scratch_shapes=[
            pltpu.VMEM((N_DEV, B, SQ, D), jnp.bfloat16),
            pltpu.VMEM((B, HQ, SQ, DH), jnp.bfloat16),
            pltpu.VMEM((HQ, B, SQ, DH), jnp.bfloat16),
            pltpu.VMEM((HQ, B, SQ, 2 * DH), jnp.bfloat16),
            pltpu.VMEM((B, HQ, SQ, 2 * DH), jnp.float32),
            pltpu.VMEM((B * SQ, D), jnp.bfloat16),
            pltpu.SemaphoreType.DMA((4,)),
            pltpu.SemaphoreType.DMA((4,)),
            pltpu.SemaphoreType.DMA((3,)),
            pltpu.SemaphoreType.DMA((3,)),
        ],
        compiler_params=pltpu.CompilerParams(
            collective_id=0, vmem_limit_bytes=63 * 1024 * 1024),
    )(ring, xb, wqb, wkb, wvb, wob, cos, sin)


# device time: 173296 ns/iter; 1.0601x vs baseline; 1.0601x over previous
import numpy as np

import jax
import jax.numpy as jnp
from jax import lax
from jax.experimental import pallas as pl
from jax.experimental.pallas import tpu as pltpu

N_DEV = 8
B = 2
SQ = 512
S_GLOBAL = N_DEV * SQ
D = 1024
HQ = 8
DH = 128
SCALE = 0.08838834764831843


def _rope(t, cos, sin):
    idx = lax.broadcasted_iota(jnp.int32, t.shape, 1)
    t_r = jnp.where(idx % 2 == 0,
                    -jnp.roll(t, -1, axis=1),
                    jnp.roll(t, 1, axis=1))
    return t * cos + t_r * sin


_NEXT = [1, 2, 3, 7, 0, 4, 5, 6]
_PREV = [4, 0, 1, 2, 5, 6, 7, 3]


def kernel(x, Wq, Wk, Wv, Wo):
    xb = x.astype(jnp.bfloat16)
    wqb = Wq.astype(jnp.bfloat16)
    wkb = Wk.astype(jnp.bfloat16)
    wvb = Wv.astype(jnp.bfloat16)
    wob = Wo.astype(jnp.bfloat16)

    inv = 1.0 / (10000.0 ** (np.arange(0, DH, 2) / DH))
    pos = (jnp.arange(S_GLOBAL, dtype=jnp.float32)[:, None]
           * jnp.asarray(inv, dtype=jnp.float32)[None, :])
    cos = jnp.repeat(jnp.cos(pos), 2, axis=-1).astype(jnp.bfloat16)
    sin = jnp.repeat(jnp.sin(pos), 2, axis=-1).astype(jnp.bfloat16)

    my_id = lax.axis_index("i")
    nxt_t = jnp.asarray(_NEXT, dtype=jnp.int32)
    prv_t = jnp.asarray(_PREV, dtype=jnp.int32)
    o_list = [my_id.astype(jnp.int32)]
    for _ in range(4):
        o_list.append(prv_t[o_list[-1]])
    n_list = [nxt_t[my_id]]
    for _ in range(2):
        n_list.append(nxt_t[n_list[-1]])
    ring = jnp.stack([nxt_t[my_id], prv_t[my_id]] + o_list + n_list)

    def body(ring_ref, x_ref, wq_ref, wk_ref, wv_ref, wo_ref, cos_ref,
             sin_ref, out_ref, x_all, q_all, k_scr, v_scr, l_ref, acc_ref,
             ctx_ref, send_sems_r, recv_sems_r, send_sems_l, recv_sems_l):
        right = ring_ref[0]
        left = ring_ref[1]

        barrier_sem = pltpu.get_barrier_semaphore()
        for nbr in (left, right):
            pl.semaphore_signal(barrier_sem, inc=1, device_id=(nbr,),
                                device_id_type=pl.DeviceIdType.MESH)
        pl.semaphore_wait(barrier_sem, 2)

        my = ring_ref[2]

        x_all[pl.ds(my, 1)] = x_ref[:][None]

        def make_rdma(t, slot, target, s_sems, r_sems):
            return pltpu.make_async_remote_copy(
                src_ref=x_all.at[slot],
                dst_ref=x_all.at[slot],
                send_sem=s_sems.at[t],
                recv_sem=r_sems.at[t],
                device_id=(target,),
                device_id_type=pl.DeviceIdType.MESH,
            )

        def mk_r(t, slot):
            return make_rdma(t, slot, right, send_sems_r, recv_sems_r)

        def mk_l(t, slot):
            return make_rdma(t, slot, left, send_sems_l, recv_sems_l)

        r0 = mk_r(0, my)
        r0.start()
        l0 = mk_l(0, my)
        l0.start()

        cos_q = cos_ref[pl.ds(my * SQ, SQ), :].astype(jnp.float32)
        sin_q = sin_ref[pl.ds(my * SQ, SQ), :].astype(jnp.float32)
        for b in range(B):
            qf = jnp.dot(x_ref[b], wq_ref[...],
                         preferred_element_type=jnp.float32)
            for h in range(HQ):
                hs = slice(h * DH, (h + 1) * DH)
                q_all[b, h] = (_rope(qf[:, hs], cos_q, sin_q)
                               * (SCALE * 1.4426950408889634)).astype(
                                   jnp.bfloat16)
        l_ref[...] = jnp.zeros((B, HQ, SQ, 1), dtype=jnp.float32)
        acc_ref[...] = jnp.zeros((B, HQ, SQ, DH), dtype=jnp.float32)

        def process(o):
            cos_o = cos_ref[pl.ds(o * SQ, SQ), :].astype(jnp.float32)
            sin_o = sin_ref[pl.ds(o * SQ, SQ), :].astype(jnp.float32)

            for b in range(B):
                xo = x_all[o, b]
                kf = jnp.dot(xo, wk_ref[...],
                             preferred_element_type=jnp.float32)
                for h in range(HQ):
                    hs = slice(h * DH, (h + 1) * DH)
                    k_scr[h, b] = _rope(kf[:, hs], cos_o, sin_o).astype(
                        jnp.bfloat16)
                vf = jnp.dot(xo, wv_ref[...],
                             preferred_element_type=jnp.float32)
                for h in range(HQ):
                    hs = slice(h * DH, (h + 1) * DH)
                    v_scr[h, b] = vf[:, hs].astype(jnp.bfloat16)

            def bh_body(i, _):
                b = i // HQ
                h = i - b * HQ
                s = lax.dot_general(
                    q_all[b, h], k_scr[h, b], (((1,), (1,)), ((), ())),
                    preferred_element_type=jnp.float32)
                p = jnp.exp2(s)
                l_ref[b, h] = l_ref[b, h] + jnp.sum(p, axis=1, keepdims=True)
                acc_ref[b, h] = acc_ref[b, h] + jnp.dot(
                    p.astype(jnp.bfloat16), v_scr[h, b],
                    preferred_element_type=jnp.float32)
                return 0

            lax.fori_loop(0, B * HQ, bh_body, 0)

        process(my)
        r0.wait()
        r1 = mk_r(1, ring_ref[3])
        r1.start()
        process(ring_ref[3])
        l0.wait()
        l1 = mk_l(1, ring_ref[7])
        l1.start()
        process(ring_ref[7])
        r1.wait()
        r2 = mk_r(2, ring_ref[4])
        r2.start()
        process(ring_ref[4])
        l1.wait()
        l2 = mk_l(2, ring_ref[8])
        l2.start()
        process(ring_ref[8])
        r2.wait()
        r3 = mk_r(3, ring_ref[5])
        r3.start()
        process(ring_ref[5])
        l2.wait()
        process(ring_ref[9])
        r3.wait()
        process(ring_ref[6])

        for b in range(B):
            for h in range(HQ):
                ctx_ref[b * SQ:(b + 1) * SQ, h * DH:(h + 1) * DH] = (
                    acc_ref[b, h] / l_ref[b, h]).astype(jnp.bfloat16)
        out = jnp.dot(ctx_ref[...], wo_ref[...],
                      preferred_element_type=jnp.float32)
        out_ref[...] = out.reshape(B, SQ, D)

    return pl.pallas_call(
        body,
        out_shape=jax.ShapeDtypeStruct((B, SQ, D), jnp.float32),
        in_specs=[pl.BlockSpec(memory_space=pltpu.SMEM)]
        + [pl.BlockSpec(memory_space=pltpu.VMEM)] * 7,
        out_specs=pl.BlockSpec(memory_space=pltpu.VMEM),
        scratch_shapes=[
            pltpu.VMEM((N_DEV, B, SQ, D), jnp.bfloat16),
            pltpu.VMEM((B, HQ, SQ, DH), jnp.bfloat16),
            pltpu.VMEM((HQ, B, SQ, DH), jnp.bfloat16),
            pltpu.VMEM((HQ, B, SQ, DH), jnp.bfloat16),
            pltpu.VMEM((B, HQ, SQ, 1), jnp.float32),
            pltpu.VMEM((B, HQ, SQ, DH), jnp.float32),
            pltpu.VMEM((B * SQ, D), jnp.bfloat16),
            pltpu.SemaphoreType.DMA((4,)),
            pltpu.SemaphoreType.DMA((4,)),
            pltpu.SemaphoreType.DMA((3,)),
            pltpu.SemaphoreType.DMA((3,)),
        ],
        compiler_params=pltpu.CompilerParams(
            collective_id=0, vmem_limit_bytes=63 * 1024 * 1024),
    )(ring, xb, wqb, wkb, wvb, wob, cos, sin)


# device time: 145559 ns/iter; 1.2622x vs baseline; 1.1906x over previous
import numpy as np

import jax
import jax.numpy as jnp
from jax import lax
from jax.experimental import pallas as pl
from jax.experimental.pallas import tpu as pltpu

N_DEV = 8
B = 2
SQ = 512
S_GLOBAL = N_DEV * SQ
D = 1024
HQ = 8
DH = 128
SCALE = 0.08838834764831843


def _rope(t, cos, sin):
    idx = lax.broadcasted_iota(jnp.int32, t.shape, 1)
    t_r = jnp.where(idx % 2 == 0,
                    -jnp.roll(t, -1, axis=1),
                    jnp.roll(t, 1, axis=1))
    return t * cos + t_r * sin


_NEXT = [1, 2, 3, 7, 0, 4, 5, 6]
_PREV = [4, 0, 1, 2, 5, 6, 7, 3]

_TABLE = []
for _d in range(N_DEV):
    _o = [_d]
    for _ in range(4):
        _o.append(_PREV[_o[-1]])
    _n = [_NEXT[_d]]
    for _ in range(2):
        _n.append(_NEXT[_n[-1]])
    _TABLE.append([_NEXT[_d], _PREV[_d]] + _o + _n)


def kernel(x, Wq, Wk, Wv, Wo):
    xb = x.astype(jnp.bfloat16)
    wqb = Wq.astype(jnp.bfloat16)
    wkb = Wk.astype(jnp.bfloat16)
    wvb = Wv.astype(jnp.bfloat16)
    wob = Wo.astype(jnp.bfloat16)

    import ml_dtypes
    inv = 1.0 / (10000.0 ** (np.arange(0, DH, 2) / DH))
    pos = np.arange(S_GLOBAL, dtype=np.float64)[:, None] * inv[None, :]
    cos = jnp.asarray(
        np.repeat(np.cos(pos), 2, axis=-1).astype(ml_dtypes.bfloat16))
    sin = jnp.asarray(
        np.repeat(np.sin(pos), 2, axis=-1).astype(ml_dtypes.bfloat16))

    ring = jnp.asarray(_TABLE, dtype=jnp.int32)

    def body(ring_ref, x_ref, wq_ref, wk_ref, wv_ref, wo_ref, cos_ref,
             sin_ref, out_ref, x_all, q_all, k_scr, v_scr, l_ref, acc_ref,
             ctx_ref, send_sems_r, recv_sems_r, send_sems_l, recv_sems_l):
        me = lax.axis_index("i")
        right = ring_ref[me, 0]
        left = ring_ref[me, 1]

        barrier_sem = pltpu.get_barrier_semaphore()
        for nbr in (left, right):
            pl.semaphore_signal(barrier_sem, inc=1, device_id=(nbr,),
                                device_id_type=pl.DeviceIdType.MESH)
        pl.semaphore_wait(barrier_sem, 2)

        my = ring_ref[me, 2]

        x_all[pl.ds(my, 1)] = x_ref[:][None]

        def make_rdma(t, slot, target, s_sems, r_sems):
            return pltpu.make_async_remote_copy(
                src_ref=x_all.at[slot],
                dst_ref=x_all.at[slot],
                send_sem=s_sems.at[t],
                recv_sem=r_sems.at[t],
                device_id=(target,),
                device_id_type=pl.DeviceIdType.MESH,
            )

        def mk_r(t, slot):
            return make_rdma(t, slot, right, send_sems_r, recv_sems_r)

        def mk_l(t, slot):
            return make_rdma(t, slot, left, send_sems_l, recv_sems_l)

        r0 = mk_r(0, my)
        r0.start()
        l0 = mk_l(0, my)
        l0.start()

        cos_q = cos_ref[pl.ds(my * SQ, SQ), :].astype(jnp.float32)
        sin_q = sin_ref[pl.ds(my * SQ, SQ), :].astype(jnp.float32)
        for b in range(B):
            qf = jnp.dot(x_ref[b], wq_ref[...],
                         preferred_element_type=jnp.float32)
            for h in range(HQ):
                hs = slice(h * DH, (h + 1) * DH)
                q_all[b, h] = (_rope(qf[:, hs], cos_q, sin_q)
                               * (SCALE * 1.4426950408889634)).astype(
                                   jnp.bfloat16)
        l_ref[...] = jnp.zeros((B, HQ, SQ, 1), dtype=jnp.float32)
        acc_ref[...] = jnp.zeros((B, HQ, SQ, DH), dtype=jnp.float32)

        def process(o):
            cos_o = cos_ref[pl.ds(o * SQ, SQ), :].astype(jnp.float32)
            sin_o = sin_ref[pl.ds(o * SQ, SQ), :].astype(jnp.float32)

            for b in range(B):
                xo = x_all[o, b]
                kf = jnp.dot(xo, wk_ref[...],
                             preferred_element_type=jnp.float32)
                for h in range(HQ):
                    hs = slice(h * DH, (h + 1) * DH)
                    k_scr[h, b] = _rope(kf[:, hs], cos_o, sin_o).astype(
                        jnp.bfloat16)
                vf = jnp.dot(xo, wv_ref[...],
                             preferred_element_type=jnp.float32)
                for h in range(HQ):
                    hs = slice(h * DH, (h + 1) * DH)
                    v_scr[h, b] = vf[:, hs].astype(jnp.bfloat16)

            def bh_body(i, _):
                b = i // HQ
                h = i - b * HQ
                s = lax.dot_general(
                    q_all[b, h], k_scr[h, b], (((1,), (1,)), ((), ())),
                    preferred_element_type=jnp.float32)
                p = jnp.exp2(s)
                l_ref[b, h] = l_ref[b, h] + jnp.sum(p, axis=1, keepdims=True)
                acc_ref[b, h] = acc_ref[b, h] + jnp.dot(
                    p.astype(jnp.bfloat16), v_scr[h, b],
                    preferred_element_type=jnp.float32)
                return 0

            lax.fori_loop(0, B * HQ, bh_body, 0)

        process(my)
        r0.wait()
        r1 = mk_r(1, ring_ref[me, 3])
        r1.start()
        process(ring_ref[me, 3])
        l0.wait()
        l1 = mk_l(1, ring_ref[me, 7])
        l1.start()
        process(ring_ref[me, 7])
        r1.wait()
        r2 = mk_r(2, ring_ref[me, 4])
        r2.start()
        process(ring_ref[me, 4])
        l1.wait()
        l2 = mk_l(2, ring_ref[me, 8])
        l2.start()
        process(ring_ref[me, 8])
        r2.wait()
        r3 = mk_r(3, ring_ref[me, 5])
        r3.start()
        process(ring_ref[me, 5])
        l2.wait()
        process(ring_ref[me, 9])
        r3.wait()
        process(ring_ref[me, 6])

        for b in range(B):
            for h in range(HQ):
                ctx_ref[b * SQ:(b + 1) * SQ, h * DH:(h + 1) * DH] = (
                    acc_ref[b, h] / l_ref[b, h]).astype(jnp.bfloat16)
        out = jnp.dot(ctx_ref[...], wo_ref[...],
                      preferred_element_type=jnp.float32)
        out_ref[...] = out.reshape(B, SQ, D)

    return pl.pallas_call(
        body,
        out_shape=jax.ShapeDtypeStruct((B, SQ, D), jnp.float32),
        in_specs=[pl.BlockSpec(memory_space=pltpu.SMEM)]
        + [pl.BlockSpec(memory_space=pltpu.VMEM)] * 7,
        out_specs=pl.BlockSpec(memory_space=pltpu.VMEM),
        scratch_shapes=[
            pltpu.VMEM((N_DEV, B, SQ, D), jnp.bfloat16),
            pltpu.VMEM((B, HQ, SQ, DH), jnp.bfloat16),
            pltpu.VMEM((HQ, B, SQ, DH), jnp.bfloat16),
            pltpu.VMEM((HQ, B, SQ, DH), jnp.bfloat16),
            pltpu.VMEM((B, HQ, SQ, 1), jnp.float32),
            pltpu.VMEM((B, HQ, SQ, DH), jnp.float32),
            pltpu.VMEM((B * SQ, D), jnp.bfloat16),
            pltpu.SemaphoreType.DMA((4,)),
            pltpu.SemaphoreType.DMA((4,)),
            pltpu.SemaphoreType.DMA((3,)),
            pltpu.SemaphoreType.DMA((3,)),
        ],
        compiler_params=pltpu.CompilerParams(
            collective_id=0, vmem_limit_bytes=63 * 1024 * 1024),
    )(ring, xb, wqb, wkb, wvb, wob, cos, sin)


# device time: 144895 ns/iter; 1.2679x vs baseline; 1.0046x over previous
import numpy as np

import jax
import jax.numpy as jnp
from jax import lax
from jax.experimental import pallas as pl
from jax.experimental.pallas import tpu as pltpu

N_DEV = 8
B = 2
SQ = 512
S_GLOBAL = N_DEV * SQ
D = 1024
HQ = 8
DH = 128
SCALE = 0.08838834764831843


def _rope(t, cos, sin):
    idx = lax.broadcasted_iota(jnp.int32, t.shape, 1)
    t_r = jnp.where(idx % 2 == 0,
                    -jnp.roll(t, -1, axis=1),
                    jnp.roll(t, 1, axis=1))
    return t * cos + t_r * sin


_NEXT = [1, 2, 3, 7, 0, 4, 5, 6]
_PREV = [4, 0, 1, 2, 5, 6, 7, 3]

_TABLE = []
for _d in range(N_DEV):
    _o = [_d]
    for _ in range(4):
        _o.append(_PREV[_o[-1]])
    _n = [_NEXT[_d]]
    for _ in range(2):
        _n.append(_NEXT[_n[-1]])
    _TABLE.append([_NEXT[_d], _PREV[_d]] + _o + _n)


def kernel(x, Wq, Wk, Wv, Wo):
    xb = x.astype(jnp.bfloat16)
    wqb = Wq.astype(jnp.bfloat16)
    wkb = Wk.astype(jnp.bfloat16)
    wvb = Wv.astype(jnp.bfloat16)
    wob = Wo.astype(jnp.bfloat16)

    import ml_dtypes
    inv = 1.0 / (10000.0 ** (np.arange(0, DH, 2) / DH))
    pos = np.arange(S_GLOBAL, dtype=np.float64)[:, None] * inv[None, :]
    cos = jnp.asarray(
        np.repeat(np.cos(pos), 2, axis=-1).astype(ml_dtypes.bfloat16))
    sin = jnp.asarray(
        np.repeat(np.sin(pos), 2, axis=-1).astype(ml_dtypes.bfloat16))

    ring = jnp.asarray(_TABLE, dtype=jnp.int32)

    def body(ring_ref, x_ref, wq_ref, wk_ref, wv_ref, wo_ref, cos_ref,
             sin_ref, out_ref, x_all, q_all, k_scr, v_scr, l_ref, acc_ref,
             ctx_ref, send_sems_r, recv_sems_r, send_sems_l, recv_sems_l):
        me = lax.axis_index("i")
        right = ring_ref[me, 0]
        left = ring_ref[me, 1]

        barrier_sem = pltpu.get_barrier_semaphore()
        for nbr in (left, right):
            pl.semaphore_signal(barrier_sem, inc=1, device_id=(nbr,),
                                device_id_type=pl.DeviceIdType.MESH)
        pl.semaphore_wait(barrier_sem, 2)

        my = ring_ref[me, 2]

        x_all[pl.ds(my, 1)] = x_ref[:][None]

        def make_rdma(t, slot, target, s_sems, r_sems):
            return pltpu.make_async_remote_copy(
                src_ref=x_all.at[slot],
                dst_ref=x_all.at[slot],
                send_sem=s_sems.at[t],
                recv_sem=r_sems.at[t],
                device_id=(target,),
                device_id_type=pl.DeviceIdType.MESH,
            )

        def mk_r(t, slot):
            return make_rdma(t, slot, right, send_sems_r, recv_sems_r)

        def mk_l(t, slot):
            return make_rdma(t, slot, left, send_sems_l, recv_sems_l)

        r0 = mk_r(0, my)
        r0.start()
        l0 = mk_l(0, my)
        l0.start()

        cos_q = cos_ref[pl.ds(my * SQ, SQ), :].astype(jnp.float32)
        sin_q = sin_ref[pl.ds(my * SQ, SQ), :].astype(jnp.float32)
        for b in range(B):
            qf = jnp.dot(x_ref[b], wq_ref[...],
                         preferred_element_type=jnp.float32)
            for h in range(HQ):
                hs = slice(h * DH, (h + 1) * DH)
                q_all[b, h] = (_rope(qf[:, hs], cos_q, sin_q)
                               * (SCALE * 1.4426950408889634)).astype(
                                   jnp.bfloat16)
        l_ref[...] = jnp.zeros((B, HQ, SQ, 1), dtype=jnp.float32)
        acc_ref[...] = jnp.zeros((B, HQ, SQ, DH), dtype=jnp.float32)

        def process(o):
            cos_o = cos_ref[pl.ds(o * SQ, SQ), :].astype(jnp.float32)
            sin_o = sin_ref[pl.ds(o * SQ, SQ), :].astype(jnp.float32)

            for b in range(B):
                xo = x_all[o, b]
                kf = jnp.dot(xo, wk_ref[...],
                             preferred_element_type=jnp.float32)
                for h in range(HQ):
                    hs = slice(h * DH, (h + 1) * DH)
                    k_scr[h, b] = _rope(kf[:, hs], cos_o, sin_o).astype(
                        jnp.bfloat16)
                vf = jnp.dot(xo, wv_ref[...],
                             preferred_element_type=jnp.float32)
                for h in range(HQ):
                    hs = slice(h * DH, (h + 1) * DH)
                    v_scr[h, b] = vf[:, hs].astype(jnp.bfloat16)

            def bh_body(i, _):
                b = i // HQ
                h = i - b * HQ
                s = lax.dot_general(
                    q_all[b, h], k_scr[h, b], (((1,), (1,)), ((), ())),
                    preferred_element_type=jnp.float32)
                p = jnp.exp2(s)
                l_ref[b, h] = l_ref[b, h] + jnp.sum(p, axis=1, keepdims=True)
                acc_ref[b, h] = acc_ref[b, h] + jnp.dot(
                    p.astype(jnp.bfloat16), v_scr[h, b],
                    preferred_element_type=jnp.float32)
                return 0

            lax.fori_loop(0, B * HQ, bh_body, 0)

        process(my)
        r0.wait()
        r1 = mk_r(1, ring_ref[me, 3])
        r1.start()
        process(ring_ref[me, 3])
        l0.wait()
        l1 = mk_l(1, ring_ref[me, 7])
        l1.start()
        process(ring_ref[me, 7])
        r1.wait()
        r2 = mk_r(2, ring_ref[me, 4])
        r2.start()
        process(ring_ref[me, 4])
        l1.wait()
        l2 = mk_l(2, ring_ref[me, 8])
        l2.start()
        process(ring_ref[me, 8])
        r2.wait()
        r3 = mk_r(3, ring_ref[me, 5])
        r3.start()
        process(ring_ref[me, 5])
        l2.wait()
        process(ring_ref[me, 9])
        r3.wait()
        process(ring_ref[me, 6])

        for b in range(B):
            for h in range(HQ):
                ctx_ref[b * SQ:(b + 1) * SQ, h * DH:(h + 1) * DH] = (
                    acc_ref[b, h] / l_ref[b, h]).astype(jnp.bfloat16)
        out = jnp.dot(ctx_ref[...], wo_ref[...],
                      preferred_element_type=jnp.float32)
        out_ref[...] = out.reshape(B, SQ, D).astype(jnp.bfloat16)

    return pl.pallas_call(
        body,
        out_shape=jax.ShapeDtypeStruct((B, SQ, D), jnp.bfloat16),
        in_specs=[pl.BlockSpec(memory_space=pltpu.SMEM)]
        + [pl.BlockSpec(memory_space=pltpu.VMEM)] * 7,
        out_specs=pl.BlockSpec(memory_space=pltpu.VMEM),
        scratch_shapes=[
            pltpu.VMEM((N_DEV, B, SQ, D), jnp.bfloat16),
            pltpu.VMEM((B, HQ, SQ, DH), jnp.bfloat16),
            pltpu.VMEM((HQ, B, SQ, DH), jnp.bfloat16),
            pltpu.VMEM((HQ, B, SQ, DH), jnp.bfloat16),
            pltpu.VMEM((B, HQ, SQ, 1), jnp.float32),
            pltpu.VMEM((B, HQ, SQ, DH), jnp.float32),
            pltpu.VMEM((B * SQ, D), jnp.bfloat16),
            pltpu.SemaphoreType.DMA((4,)),
            pltpu.SemaphoreType.DMA((4,)),
            pltpu.SemaphoreType.DMA((3,)),
            pltpu.SemaphoreType.DMA((3,)),
        ],
        compiler_params=pltpu.CompilerParams(
            collective_id=0, vmem_limit_bytes=63 * 1024 * 1024),
    )(ring, xb, wqb, wkb, wvb, wob, cos, sin)


# device time: 138768 ns/iter; 1.3239x vs baseline; 1.0442x over previous
import numpy as np

import jax
import jax.numpy as jnp
from jax import lax
from jax.experimental import pallas as pl
from jax.experimental.pallas import tpu as pltpu

N_DEV = 8
B = 2
SQ = 512
S_GLOBAL = N_DEV * SQ
D = 1024
HQ = 8
DH = 128
SCALE = 0.08838834764831843


def _rope(t, cos, sin):
    idx = lax.broadcasted_iota(jnp.int32, t.shape, 1)
    t_r = jnp.where(idx % 2 == 0,
                    -jnp.roll(t, -1, axis=1),
                    jnp.roll(t, 1, axis=1))
    return t * cos + t_r * sin


_NEXT = [1, 2, 3, 7, 0, 4, 5, 6]
_PREV = [4, 0, 1, 2, 5, 6, 7, 3]

_TABLE = []
for _d in range(N_DEV):
    _o = [_d]
    for _ in range(4):
        _o.append(_PREV[_o[-1]])
    _n = [_NEXT[_d]]
    for _ in range(2):
        _n.append(_NEXT[_n[-1]])
    _TABLE.append([_NEXT[_d], _PREV[_d]] + _o + _n)


def kernel(x, Wq, Wk, Wv, Wo):
    xb = x.astype(jnp.bfloat16)
    wqb = Wq.astype(jnp.bfloat16)
    wkb = Wk.astype(jnp.bfloat16)
    wvb = Wv.astype(jnp.bfloat16)
    wob = Wo.astype(jnp.bfloat16)

    import ml_dtypes
    inv = 1.0 / (10000.0 ** (np.arange(0, DH, 2) / DH))
    pos = np.arange(S_GLOBAL, dtype=np.float64)[:, None] * inv[None, :]
    cos = jnp.asarray(
        np.repeat(np.cos(pos), 2, axis=-1).astype(ml_dtypes.bfloat16))
    sin = jnp.asarray(
        np.repeat(np.sin(pos), 2, axis=-1).astype(ml_dtypes.bfloat16))

    ring = jnp.asarray(_TABLE, dtype=jnp.int32)

    def body(ring_ref, x_ref, wq_ref, wk_ref, wv_ref, wo_ref, cos_ref,
             sin_ref, out_ref, x_all, q_all, k_scr, v_scr, l_ref, acc_ref,
             ctx_ref, send_sems_r, recv_sems_r, send_sems_l, recv_sems_l):
        me = lax.axis_index("i")
        right = ring_ref[me, 0]
        left = ring_ref[me, 1]

        barrier_sem = pltpu.get_barrier_semaphore()
        for nbr in (left, right):
            pl.semaphore_signal(barrier_sem, inc=1, device_id=(nbr,),
                                device_id_type=pl.DeviceIdType.MESH)
        pl.semaphore_wait(barrier_sem, 2)

        my = ring_ref[me, 2]

        x_all[pl.ds(my, 1)] = x_ref[:][None]

        def make_rdma(t, slot, target, s_sems, r_sems):
            return pltpu.make_async_remote_copy(
                src_ref=x_all.at[slot],
                dst_ref=x_all.at[slot],
                send_sem=s_sems.at[t],
                recv_sem=r_sems.at[t],
                device_id=(target,),
                device_id_type=pl.DeviceIdType.MESH,
            )

        def mk_r(t, slot):
            return make_rdma(t, slot, right, send_sems_r, recv_sems_r)

        def mk_l(t, slot):
            return make_rdma(t, slot, left, send_sems_l, recv_sems_l)

        r0 = mk_r(0, my)
        r0.start()
        l0 = mk_l(0, my)
        l0.start()

        cos_q = cos_ref[pl.ds(my * SQ, SQ), :].astype(jnp.float32)
        sin_q = sin_ref[pl.ds(my * SQ, SQ), :].astype(jnp.float32)
        for b in range(B):
            qf = jnp.dot(x_ref[b], wq_ref[...],
                         preferred_element_type=jnp.float32)
            for h in range(HQ):
                hs = slice(h * DH, (h + 1) * DH)
                q_all[b, h] = (_rope(qf[:, hs], cos_q, sin_q)
                               * (SCALE * 1.4426950408889634)).astype(
                                   jnp.bfloat16)
        l_ref[...] = jnp.zeros((B, HQ, SQ, 1), dtype=jnp.float32)
        acc_ref[...] = jnp.zeros((B, HQ, SQ, DH), dtype=jnp.float32)

        def stage(o, j):
            cos_o = cos_ref[pl.ds(o * SQ, SQ), :].astype(jnp.float32)
            sin_o = sin_ref[pl.ds(o * SQ, SQ), :].astype(jnp.float32)
            for b in range(B):
                xo = x_all[o, b]
                kf = jnp.dot(xo, wk_ref[...],
                             preferred_element_type=jnp.float32)
                for h in range(HQ):
                    hs = slice(h * DH, (h + 1) * DH)
                    k_scr[h, b, j] = _rope(kf[:, hs], cos_o, sin_o).astype(
                        jnp.bfloat16)
                vf = jnp.dot(xo, wv_ref[...],
                             preferred_element_type=jnp.float32)
                for h in range(HQ):
                    hs = slice(h * DH, (h + 1) * DH)
                    v_scr[h, b, j] = vf[:, hs].astype(jnp.bfloat16)

        def attn(n_blocks):
            nk = n_blocks * SQ

            def bh_body(i, _):
                b = i // HQ
                h = i - b * HQ
                k2 = k_scr[h, b, 0:n_blocks].reshape(nk, DH)
                v2 = v_scr[h, b, 0:n_blocks].reshape(nk, DH)
                s = lax.dot_general(
                    q_all[b, h], k2, (((1,), (1,)), ((), ())),
                    preferred_element_type=jnp.float32)
                p = jnp.exp2(s)
                l_ref[b, h] = l_ref[b, h] + jnp.sum(p, axis=1, keepdims=True)
                acc_ref[b, h] = acc_ref[b, h] + jnp.dot(
                    p.astype(jnp.bfloat16), v2,
                    preferred_element_type=jnp.float32)
                return 0

            lax.fori_loop(0, B * HQ, bh_body, 0)

        stage(my, 0)
        attn(1)
        r0.wait()
        r1 = mk_r(1, ring_ref[me, 3])
        r1.start()
        l0.wait()
        l1 = mk_l(1, ring_ref[me, 7])
        l1.start()
        stage(ring_ref[me, 3], 0)
        stage(ring_ref[me, 7], 1)
        attn(2)
        r1.wait()
        r2 = mk_r(2, ring_ref[me, 4])
        r2.start()
        l1.wait()
        l2 = mk_l(2, ring_ref[me, 8])
        l2.start()
        stage(ring_ref[me, 4], 0)
        stage(ring_ref[me, 8], 1)
        attn(2)
        r2.wait()
        r3 = mk_r(3, ring_ref[me, 5])
        r3.start()
        l2.wait()
        stage(ring_ref[me, 5], 0)
        stage(ring_ref[me, 9], 1)
        attn(2)
        r3.wait()
        stage(ring_ref[me, 6], 0)
        attn(1)

        for b in range(B):
            for h in range(HQ):
                ctx_ref[b * SQ:(b + 1) * SQ, h * DH:(h + 1) * DH] = (
                    acc_ref[b, h] / l_ref[b, h]).astype(jnp.bfloat16)
        out = jnp.dot(ctx_ref[...], wo_ref[...],
                      preferred_element_type=jnp.float32)
        out_ref[...] = out.reshape(B, SQ, D).astype(jnp.bfloat16)

    return pl.pallas_call(
        body,
        out_shape=jax.ShapeDtypeStruct((B, SQ, D), jnp.bfloat16),
        in_specs=[pl.BlockSpec(memory_space=pltpu.SMEM)]
        + [pl.BlockSpec(memory_space=pltpu.VMEM)] * 7,
        out_specs=pl.BlockSpec(memory_space=pltpu.VMEM),
        scratch_shapes=[
            pltpu.VMEM((N_DEV, B, SQ, D), jnp.bfloat16),
            pltpu.VMEM((B, HQ, SQ, DH), jnp.bfloat16),
            pltpu.VMEM((HQ, B, 2, SQ, DH), jnp.bfloat16),
            pltpu.VMEM((HQ, B, 2, SQ, DH), jnp.bfloat16),
            pltpu.VMEM((B, HQ, SQ, 1), jnp.float32),
            pltpu.VMEM((B, HQ, SQ, DH), jnp.float32),
            pltpu.VMEM((B * SQ, D), jnp.bfloat16),
            pltpu.SemaphoreType.DMA((4,)),
            pltpu.SemaphoreType.DMA((4,)),
            pltpu.SemaphoreType.DMA((3,)),
            pltpu.SemaphoreType.DMA((3,)),
        ],
        compiler_params=pltpu.CompilerParams(
            collective_id=0, vmem_limit_bytes=63 * 1024 * 1024),
    )(ring, xb, wqb, wkb, wvb, wob, cos, sin)
